# baseline (speedup 1.0000x reference)
import functools

import jax
import jax.numpy as jnp
from jax import lax
from jax.experimental import pallas as pl
from jax.experimental.pallas import tpu as pltpu
from jax.experimental.pallas import tpu_sc as plsc

T = 2048
D = 1024
DFF = 2816
NE = 8
EPAD = 128

TM = 128
NB = 40
GPAD = NB * TM
TFF = 256
NF = DFF // TFF

NC = 2
NS = 16
NW = NC * NS


def _router_body(x_ref, w_ref, b_ref, e1_ref, e2_ref, w1_ref, w2_ref):
    x = x_ref[...]
    w = w_ref[...]
    logits = lax.dot_general(x, w, (((1,), (1,)), ((), ())),
                             preferred_element_type=jnp.float32)
    logits = logits + b_ref[...]
    eidx = lax.broadcasted_iota(jnp.int32, (T, EPAD), 1)
    m1 = jnp.max(logits, axis=1, keepdims=True)
    e1 = jnp.min(jnp.where(logits >= m1, eidx, EPAD), axis=1, keepdims=True)
    l2 = jnp.where(eidx == e1, -1e30, logits)
    m2 = jnp.max(l2, axis=1, keepdims=True)
    e2 = jnp.min(jnp.where(l2 >= m2, eidx, EPAD), axis=1, keepdims=True)
    e1_ref[...] = e1
    e2_ref[...] = e2
    w1_ref[...] = jax.nn.sigmoid(m1 - m2)
    w2_ref[...] = jax.nn.sigmoid(m2 - m1)


def _run_router(flat, switch_w, switch_b):
    wpad = jnp.zeros((EPAD, D), jnp.float32).at[:NE].set(switch_w)
    bpad = jnp.full((1, EPAD), -1e30, jnp.float32).at[0, :NE].set(switch_b)
    return pl.pallas_call(
        _router_body,
        out_shape=(
            jax.ShapeDtypeStruct((T, 1), jnp.int32),
            jax.ShapeDtypeStruct((T, 1), jnp.int32),
            jax.ShapeDtypeStruct((T, 1), jnp.float32),
            jax.ShapeDtypeStruct((T, 1), jnp.float32),
        ),
    )(flat, wpad, bpad)


_GCH = 32
_GROWS = GPAD // NW


def _sc_mesh():
    return plsc.VectorSubcoreMesh(core_axis_name="c", subcore_axis_name="s",
                                  num_cores=NC, num_subcores=NS)


@functools.cache
def _make_sc_gather():
    @functools.partial(
        pl.kernel,
        out_type=jax.ShapeDtypeStruct((GPAD, D), jnp.float32),
        mesh=_sc_mesh(),
        scratch_types=[
            pltpu.VMEM((_GCH,), jnp.int32),
            pltpu.VMEM((_GCH, D), jnp.float32),
            pltpu.SemaphoreType.DMA,
        ],
    )
    def _sc_gather(x_hbm, idx_hbm, out_hbm, idx_v, rows_v, sem):
        c = lax.axis_index("c")
        s = lax.axis_index("s")
        base = (c * NS + s) * _GROWS

        def chunk(i, carry):
            off = base + i * _GCH
            pltpu.sync_copy(idx_hbm.at[pl.ds(off, _GCH)], idx_v)
            pltpu.async_copy(x_hbm.at[idx_v], rows_v, sem).wait()
            pltpu.sync_copy(rows_v, out_hbm.at[pl.ds(off, _GCH)])
            return carry

        lax.fori_loop(0, _GROWS // _GCH, chunk, 0)

    return _sc_gather


def _ffn_body(blk_ref, val_ref, x_ref, g_ref, u_ref, d_ref, cw_ref, y_ref,
              acc_ref):
    b = pl.program_id(0)
    f = pl.program_id(1)

    @pl.when(val_ref[b] != 0)
    def _():
        xb = x_ref[...]
        g = lax.dot_general(xb, g_ref[0], (((1,), (1,)), ((), ())),
                            preferred_element_type=jnp.float32)
        u = lax.dot_general(xb, u_ref[0], (((1,), (1,)), ((), ())),
                            preferred_element_type=jnp.float32)
        p = g * jax.nn.sigmoid(g) * u
        contrib = lax.dot_general(p, d_ref[0], (((1,), (1,)), ((), ())),
                                  preferred_element_type=jnp.float32)

        @pl.when(f == 0)
        def _():
            acc_ref[...] = contrib

        @pl.when(f != 0)
        def _():
            acc_ref[...] += contrib

        @pl.when(f == NF - 1)
        def _():
            y_ref[...] = acc_ref[...] * cw_ref[...]


def _run_ffn(sorted_x, gate_w, up_w, down_w, cw_col, blk_e, blk_valid):
    grid_spec = pltpu.PrefetchScalarGridSpec(
        num_scalar_prefetch=2,
        grid=(NB, NF),
        in_specs=[
            pl.BlockSpec((TM, D), lambda b, f, blk, val: (b, 0)),
            pl.BlockSpec((1, TFF, D), lambda b, f, blk, val: (blk[b], f, 0)),
            pl.BlockSpec((1, TFF, D), lambda b, f, blk, val: (blk[b], f, 0)),
            pl.BlockSpec((1, D, TFF), lambda b, f, blk, val: (blk[b], 0, f)),
            pl.BlockSpec((TM, 1), lambda b, f, blk, val: (b, 0)),
        ],
        out_specs=pl.BlockSpec((TM, D), lambda b, f, blk, val: (b, 0)),
        scratch_shapes=[pltpu.VMEM((TM, D), jnp.float32)],
    )
    return pl.pallas_call(
        _ffn_body,
        grid_spec=grid_spec,
        out_shape=jax.ShapeDtypeStruct((GPAD, D), jnp.float32),
        compiler_params=pltpu.CompilerParams(
            dimension_semantics=("arbitrary", "arbitrary")),
    )(blk_e, blk_valid, sorted_x, gate_w, up_w, down_w, cw_col)


_CCH = 32
_CTOK = T // NW


@functools.cache
def _make_sc_combine():
    @functools.partial(
        pl.kernel,
        out_type=jax.ShapeDtypeStruct((T, D), jnp.float32),
        mesh=_sc_mesh(),
        scratch_types=[
            pltpu.VMEM((_CCH,), jnp.int32),
            pltpu.VMEM((_CCH, D), jnp.float32),
            pltpu.VMEM((_CCH, D), jnp.float32),
            pltpu.SemaphoreType.DMA,
        ],
    )
    def _sc_combine(y_hbm, d1_hbm, d2_hbm, out_hbm, idx_v, rows_v, acc_v,
                    sem):
        c = lax.axis_index("c")
        s = lax.axis_index("s")
        goff = (c * NS + s) * _CTOK

        def chunk(i, carry):
            og = goff + i * _CCH
            pltpu.sync_copy(d1_hbm.at[pl.ds(og, _CCH)], idx_v)
            pltpu.async_copy(y_hbm.at[idx_v], acc_v, sem).wait()
            pltpu.sync_copy(d2_hbm.at[pl.ds(og, _CCH)], idx_v)
            pltpu.async_copy(y_hbm.at[idx_v], rows_v, sem).wait()

            def row_add(r, carry2):
                for k in range(D // 16):
                    sl = pl.ds(16 * k, 16)
                    acc_v[r, sl] += rows_v[r, sl]
                return carry2

            lax.fori_loop(0, _CCH, row_add, 0)
            pltpu.sync_copy(acc_v, out_hbm.at[pl.ds(og, _CCH)])
            return carry

        lax.fori_loop(0, _CTOK // _CCH, chunk, 0)

    return _sc_combine


def _sc_gather(flat, src_tok):
    return _make_sc_gather()(flat, src_tok)


def _sc_combine(y, d1, d2):
    return _make_sc_combine()(y, d1, d2)


def kernel(x, switch_w, switch_b, gate_w, up_w, down_w):
    bsz, n, d = x.shape
    flat = x.reshape(-1, d)

    e1, e2, w1, w2 = _run_router(flat, switch_w, switch_b)
    e1 = e1[:, 0]
    e2 = e2[:, 0]

    a = jnp.concatenate([e1, e2])
    oh = jax.nn.one_hot(a, NE, dtype=jnp.int32)
    ranks = jnp.cumsum(oh, axis=0) - oh
    rank = jnp.take_along_axis(ranks, a[:, None], axis=1)[:, 0]
    counts = jnp.sum(oh, axis=0)
    padded = ((counts + TM - 1) // TM) * TM
    cum_end = jnp.cumsum(padded)
    pad_off = cum_end - padded
    dest = pad_off[a] + rank
    used = cum_end[NE - 1]

    block_starts = jnp.arange(NB, dtype=jnp.int32) * TM
    blk_e = jnp.minimum(
        jnp.searchsorted(cum_end, block_starts, side="right"),
        NE - 1).astype(jnp.int32)
    valid = (block_starts < used)
    last_e = blk_e[jnp.maximum(used // TM - 1, 0)]
    blk_e = jnp.where(valid, blk_e, last_e)
    blk_valid = valid.astype(jnp.int32)

    tok = jnp.concatenate([jnp.arange(T, dtype=jnp.int32)] * 2)
    src_tok = jnp.zeros((GPAD,), jnp.int32).at[dest].set(tok)
    cw_assign = jnp.concatenate([w1[:, 0], w2[:, 0]])
    cw_col = jnp.zeros((GPAD, 1), jnp.float32).at[dest, 0].set(cw_assign)

    sorted_x = _sc_gather(flat, src_tok)
    y = _run_ffn(sorted_x, gate_w, up_w, down_w, cw_col, blk_e, blk_valid)
    out = _sc_combine(y, dest[:T].astype(jnp.int32),
                      dest[T:].astype(jnp.int32))
    return out.reshape(bsz, n, d)


# speedup vs baseline: 1.2063x; 1.2063x over previous
import functools

import jax
import jax.numpy as jnp
from jax import lax
from jax.experimental import pallas as pl
from jax.experimental.pallas import tpu as pltpu
from jax.experimental.pallas import tpu_sc as plsc

T = 2048
D = 1024
DFF = 2816
NE = 8
EPAD = 128

TM = 128
NB = 40
GPAD = NB * TM
TFF = 256
NF = DFF // TFF

NC = 2
NS = 16
NW = NC * NS


def _router_body(x_ref, w_ref, b_ref, e1_ref, e2_ref, w1_ref, w2_ref):
    x = x_ref[...]
    w = w_ref[...]
    logits = lax.dot_general(x, w, (((1,), (1,)), ((), ())),
                             preferred_element_type=jnp.float32)
    logits = logits + b_ref[...]
    eidx = lax.broadcasted_iota(jnp.int32, (T, EPAD), 1)
    m1 = jnp.max(logits, axis=1, keepdims=True)
    e1 = jnp.min(jnp.where(logits >= m1, eidx, EPAD), axis=1, keepdims=True)
    l2 = jnp.where(eidx == e1, -1e30, logits)
    m2 = jnp.max(l2, axis=1, keepdims=True)
    e2 = jnp.min(jnp.where(l2 >= m2, eidx, EPAD), axis=1, keepdims=True)
    e1_ref[...] = e1
    e2_ref[...] = e2
    w1_ref[...] = jax.nn.sigmoid(m1 - m2)
    w2_ref[...] = jax.nn.sigmoid(m2 - m1)


def _run_router(flat, switch_w, switch_b):
    wpad = jnp.zeros((EPAD, D), jnp.float32).at[:NE].set(switch_w)
    bpad = jnp.full((1, EPAD), -1e30, jnp.float32).at[0, :NE].set(switch_b)
    return pl.pallas_call(
        _router_body,
        out_shape=(
            jax.ShapeDtypeStruct((T, 1), jnp.int32),
            jax.ShapeDtypeStruct((T, 1), jnp.int32),
            jax.ShapeDtypeStruct((T, 1), jnp.float32),
            jax.ShapeDtypeStruct((T, 1), jnp.float32),
        ),
    )(flat, wpad, bpad)


_GCH = 32
_GROWS = GPAD // NW


def _sc_mesh():
    return plsc.VectorSubcoreMesh(core_axis_name="c", subcore_axis_name="s",
                                  num_cores=NC, num_subcores=NS)


@functools.cache
def _make_sc_gather():
    @functools.partial(
        pl.kernel,
        out_type=jax.ShapeDtypeStruct((GPAD, D), jnp.float32),
        mesh=_sc_mesh(),
        scratch_types=[
            pltpu.VMEM((_GCH,), jnp.int32),
            pltpu.VMEM((_GCH, D), jnp.float32),
            pltpu.SemaphoreType.DMA,
        ],
    )
    def _sc_gather(x_hbm, idx_hbm, out_hbm, idx_v, rows_v, sem):
        c = lax.axis_index("c")
        s = lax.axis_index("s")
        base = (c * NS + s) * _GROWS

        def chunk(i, carry):
            off = base + i * _GCH
            pltpu.sync_copy(idx_hbm.at[pl.ds(off, _GCH)], idx_v)
            pltpu.async_copy(x_hbm.at[idx_v], rows_v, sem).wait()
            pltpu.sync_copy(rows_v, out_hbm.at[pl.ds(off, _GCH)])
            return carry

        lax.fori_loop(0, _GROWS // _GCH, chunk, 0)

    return _sc_gather


def _ffn_body(blk_ref, val_ref, x_ref, g_ref, u_ref, d_ref, cw_ref, y_ref):
    f = pl.program_id(0)
    b = pl.program_id(1)

    @pl.when(val_ref[b] != 0)
    def _():
        rows = pl.ds(b * TM, TM)
        xb = x_ref[rows, :]
        g = lax.dot_general(xb, g_ref[0], (((1,), (1,)), ((), ())),
                            preferred_element_type=jnp.float32)
        u = lax.dot_general(xb, u_ref[0], (((1,), (1,)), ((), ())),
                            preferred_element_type=jnp.float32)
        p = g * jax.nn.sigmoid(g) * u
        contrib = lax.dot_general(p, d_ref[0], (((1,), (1,)), ((), ())),
                                  preferred_element_type=jnp.float32)

        @pl.when(f == 0)
        def _():
            y_ref[rows, :] = contrib

        @pl.when(f != 0)
        def _():
            y_ref[rows, :] += contrib

        @pl.when(f == NF - 1)
        def _():
            y_ref[rows, :] *= cw_ref[...]


def _run_ffn(sorted_x, gate_w, up_w, down_w, cw_col, blk_e, blk_valid):
    grid_spec = pltpu.PrefetchScalarGridSpec(
        num_scalar_prefetch=2,
        grid=(NF, NB),
        in_specs=[
            pl.BlockSpec((GPAD, D), lambda f, b, blk, val: (0, 0)),
            pl.BlockSpec((1, TFF, D), lambda f, b, blk, val: (blk[b], f, 0)),
            pl.BlockSpec((1, TFF, D), lambda f, b, blk, val: (blk[b], f, 0)),
            pl.BlockSpec((1, D, TFF), lambda f, b, blk, val: (blk[b], 0, f)),
            pl.BlockSpec((TM, 1), lambda f, b, blk, val: (b, 0)),
        ],
        out_specs=pl.BlockSpec((GPAD, D), lambda f, b, blk, val: (0, 0)),
    )
    return pl.pallas_call(
        _ffn_body,
        grid_spec=grid_spec,
        out_shape=jax.ShapeDtypeStruct((GPAD, D), jnp.float32),
        compiler_params=pltpu.CompilerParams(
            dimension_semantics=("arbitrary", "arbitrary")),
    )(blk_e, blk_valid, sorted_x, gate_w, up_w, down_w, cw_col)


_CCH = 32
_CTOK = T // NW


@functools.cache
def _make_sc_combine():
    @functools.partial(
        pl.kernel,
        out_type=jax.ShapeDtypeStruct((T, D), jnp.float32),
        mesh=_sc_mesh(),
        scratch_types=[
            pltpu.VMEM((_CCH,), jnp.int32),
            pltpu.VMEM((_CCH, D), jnp.float32),
            pltpu.VMEM((_CCH, D), jnp.float32),
            pltpu.SemaphoreType.DMA,
        ],
    )
    def _sc_combine(y_hbm, d1_hbm, d2_hbm, out_hbm, idx_v, rows_v, acc_v,
                    sem):
        c = lax.axis_index("c")
        s = lax.axis_index("s")
        goff = (c * NS + s) * _CTOK

        def chunk(i, carry):
            og = goff + i * _CCH
            pltpu.sync_copy(d1_hbm.at[pl.ds(og, _CCH)], idx_v)
            pltpu.async_copy(y_hbm.at[idx_v], acc_v, sem).wait()
            pltpu.sync_copy(d2_hbm.at[pl.ds(og, _CCH)], idx_v)
            pltpu.async_copy(y_hbm.at[idx_v], rows_v, sem).wait()

            def row_add(r, carry2):
                for k in range(D // 16):
                    sl = pl.ds(16 * k, 16)
                    acc_v[r, sl] += rows_v[r, sl]
                return carry2

            lax.fori_loop(0, _CCH, row_add, 0)
            pltpu.sync_copy(acc_v, out_hbm.at[pl.ds(og, _CCH)])
            return carry

        lax.fori_loop(0, _CTOK // _CCH, chunk, 0)

    return _sc_combine


def _sc_gather(flat, src_tok):
    return _make_sc_gather()(flat, src_tok)


def _sc_combine(y, d1, d2):
    return _make_sc_combine()(y, d1, d2)


def kernel(x, switch_w, switch_b, gate_w, up_w, down_w):
    bsz, n, d = x.shape
    flat = x.reshape(-1, d)

    e1, e2, w1, w2 = _run_router(flat, switch_w, switch_b)
    e1 = e1[:, 0]
    e2 = e2[:, 0]

    a = jnp.concatenate([e1, e2])
    oh = jax.nn.one_hot(a, NE, dtype=jnp.int32)
    ranks = jnp.cumsum(oh, axis=0) - oh
    rank = jnp.take_along_axis(ranks, a[:, None], axis=1)[:, 0]
    counts = jnp.sum(oh, axis=0)
    padded = ((counts + TM - 1) // TM) * TM
    cum_end = jnp.cumsum(padded)
    pad_off = cum_end - padded
    dest = pad_off[a] + rank
    used = cum_end[NE - 1]

    block_starts = jnp.arange(NB, dtype=jnp.int32) * TM
    blk_e = jnp.minimum(
        jnp.searchsorted(cum_end, block_starts, side="right"),
        NE - 1).astype(jnp.int32)
    valid = (block_starts < used)
    last_e = blk_e[jnp.maximum(used // TM - 1, 0)]
    blk_e = jnp.where(valid, blk_e, last_e)
    blk_valid = valid.astype(jnp.int32)

    tok = jnp.concatenate([jnp.arange(T, dtype=jnp.int32)] * 2)
    src_tok = jnp.zeros((GPAD,), jnp.int32).at[dest].set(tok)
    cw_assign = jnp.concatenate([w1[:, 0], w2[:, 0]])
    cw_col = jnp.zeros((GPAD, 1), jnp.float32).at[dest, 0].set(cw_assign)

    sorted_x = _sc_gather(flat, src_tok)
    y = _run_ffn(sorted_x, gate_w, up_w, down_w, cw_col, blk_e, blk_valid)
    out = _sc_combine(y, dest[:T].astype(jnp.int32),
                      dest[T:].astype(jnp.int32))
    return out.reshape(bsz, n, d)
